# baseline (speedup 1.0000x reference)
import jax
import jax.numpy as jnp
from jax import lax
from jax.experimental import pallas as pl
from jax.experimental.pallas import tpu as pltpu
from jax.experimental.pallas import tpu_sc as plsc

N_ATOMS = 100000
N_PAIRS = 6400000
NPAD = 100352
K = 128
NW = 32
CHUNKS_PER_W = 1564
NH = CHUNKS_PER_W // 2
P_PAD = NW * CHUNKS_PER_W * K
ROWS_PER_TILE = NPAD // 16
STAGE_BLOCKS = ROWS_PER_TILE // K
R128 = (8 * NPAD) // 128
REAL_ROWS = (8 * N_ATOMS) // 128


def _rsqrt(x):
    i = plsc.bitcast(x, jnp.int32)
    i = 0x5F3759DF - lax.shift_right_logical(i, 1)
    y = plsc.bitcast(i, jnp.float32)
    for _ in range(3):
        y = y * (1.5 - 0.5 * x * y * y)
    return y


def _sc_body(pos_hbm, ii_hbm, jj_hbm, part_hbm,
             pos_s, acc_s,
             idx_i0, idx_j0, gi0, gj0, ui0, uj0,
             idx_i1, idx_j1, gi1, gj1, ui1, uj1,
             stage_f, gsem0, gsem1, ssem0, ssem1):
    c = lax.axis_index("c")
    s = lax.axis_index("s")

    IDX_I = (idx_i0, idx_i1)
    IDX_J = (idx_j0, idx_j1)
    GI = (gi0, gi1)
    GJ = (gj0, gj1)
    UI = (ui0, ui1)
    UJ = (uj0, uj1)
    GSEM = (gsem0, gsem1)
    SSEM = (ssem0, ssem1)

    lane = lax.iota(jnp.int32, 16)
    row2 = lax.shift_right_logical(lane, 3)
    col8 = lane & 7
    zeros16 = jnp.zeros((16,), jnp.float32)

    def _zero_buf(buf):
        def zb(g, carry):
            plsc.store_scatter(buf, [g * 2 + row2, col8], zeros16)
            return carry
        lax.fori_loop(0, K * 8 // 16, zb, 0)

    for buf in (ui0, uj0, ui1, uj1):
        _zero_buf(buf)

    r0 = s * ROWS_PER_TILE

    def _to_rows(g, carry):
        v = stage_f[pl.ds(g * 16, 16)]
        plsc.store_scatter(gi0, [g * 2 + row2, col8], v)
        return carry

    def stage_block(blk, carry):
        b0 = r0 + blk * K
        pltpu.sync_copy(pos_hbm.at[pl.ds(b0 * 8, K * 8)], stage_f)
        lax.fori_loop(0, K * 8 // 16, _to_rows, 0)
        pltpu.sync_copy(gi0, pos_s.at[pl.ds(b0, K)])
        pltpu.sync_copy(ui0, acc_s.at[pl.ds(b0, K)])
        return carry

    lax.fori_loop(0, STAGE_BLOCKS, stage_block, 0)
    plsc.subcore_barrier()

    w = s * 2 + c
    base = w * (CHUNKS_PER_W * K)
    col0 = jnp.full((16,), 0, jnp.int32)
    col1 = jnp.full((16,), 1, jnp.int32)
    col2 = jnp.full((16,), 2, jnp.int32)
    col3 = jnp.full((16,), 3, jnp.int32)

    def fetch(b, off):
        pltpu.sync_copy(ii_hbm.at[pl.ds(off, K)], IDX_I[b])
        pltpu.sync_copy(jj_hbm.at[pl.ds(off, K)], IDX_J[b])
        pltpu.async_copy(pos_s.at[IDX_I[b]], GI[b], GSEM[b])
        pltpu.async_copy(pos_s.at[IDX_J[b]], GJ[b], GSEM[b])

    def wait_gathers(b):
        pltpu.make_async_copy(pos_s.at[IDX_I[b]], GI[b], GSEM[b]).wait()
        pltpu.make_async_copy(pos_s.at[IDX_J[b]], GJ[b], GSEM[b]).wait()

    def fire_scatters(b):
        pltpu.async_copy(UI[b], acc_s.at[IDX_I[b]], SSEM[b], add=True)
        pltpu.async_copy(UJ[b], acc_s.at[IDX_J[b]], SSEM[b], add=True)

    def wait_scatters(b):
        pltpu.make_async_copy(UI[b], acc_s.at[IDX_I[b]], SSEM[b]).wait()
        pltpu.make_async_copy(UJ[b], acc_s.at[IDX_J[b]], SSEM[b]).wait()

    def compute(b):
        gi_v, gj_v, ui_v, uj_v = GI[b], GJ[b], UI[b], UJ[b]
        for grp in range(K // 16):
            rows = grp * 16 + lane
            xi = plsc.load_gather(gi_v, [rows, col0])
            yi = plsc.load_gather(gi_v, [rows, col1])
            zi = plsc.load_gather(gi_v, [rows, col2])
            xj = plsc.load_gather(gj_v, [rows, col0])
            yj = plsc.load_gather(gj_v, [rows, col1])
            zj = plsc.load_gather(gj_v, [rows, col2])
            dx = xj - xi
            dy = yj - yi
            dz = zj - zi
            sq = jnp.maximum(dx * dx + dy * dy + dz * dz, 1e-24)
            yv = _rsqrt(sq)
            dist = sq * yv
            t = jnp.maximum(1.0 - dist, 0.0)
            inv_d = t * yv
            fx = inv_d * dx
            fy = inv_d * dy
            fz = inv_d * dz
            he = 0.25 * t * t
            plsc.store_scatter(ui_v, [rows, col0], fx)
            plsc.store_scatter(ui_v, [rows, col1], fy)
            plsc.store_scatter(ui_v, [rows, col2], fz)
            plsc.store_scatter(ui_v, [rows, col3], he)
            plsc.store_scatter(uj_v, [rows, col0], -fx)
            plsc.store_scatter(uj_v, [rows, col1], -fy)
            plsc.store_scatter(uj_v, [rows, col2], -fz)
            plsc.store_scatter(uj_v, [rows, col3], he)

    fetch(0, base)

    def hbody(h, carry):
        wait_gathers(0)
        compute(0)

        @pl.when(h >= 1)
        def _():
            wait_scatters(1)
        fire_scatters(0)
        fetch(1, base + (2 * h + 1) * K)

        wait_gathers(1)
        compute(1)
        wait_scatters(0)
        fire_scatters(1)

        @pl.when(h < NH - 1)
        def _():
            fetch(0, base + (2 * h + 2) * K)
        return carry

    lax.fori_loop(0, NH, hbody, 0)
    wait_scatters(1)
    plsc.subcore_barrier()

    def _from_rows(g, carry):
        v = plsc.load_gather(gi0, [g * 2 + row2, col8])
        stage_f[pl.ds(g * 16, 16)] = v
        return carry

    def out_block(blk, carry):
        b0 = r0 + blk * K
        pltpu.sync_copy(acc_s.at[pl.ds(b0, K)], gi0)
        lax.fori_loop(0, K * 8 // 16, _from_rows, 0)
        pltpu.sync_copy(stage_f,
                        part_hbm.at[pl.ds(c * 8 * NPAD + b0 * 8, K * 8)])
        return carry

    lax.fori_loop(0, STAGE_BLOCKS, out_block, 0)


@jax.jit
def _sc_call(pos8f, ii, jj):
    mesh = plsc.VectorSubcoreMesh(core_axis_name="c", subcore_axis_name="s")
    table = pltpu.VMEM_SHARED((NPAD, 8), jnp.float32)
    fbuf = pltpu.VMEM((K, 8), jnp.float32)
    ibuf = pltpu.VMEM((K,), jnp.int32)
    bufset = [ibuf, ibuf, fbuf, fbuf, fbuf, fbuf]
    return pl.kernel(
        _sc_body,
        out_type=jax.ShapeDtypeStruct((2 * 8 * NPAD,), jnp.float32),
        mesh=mesh,
        scratch_types=(
            [table, table] + bufset + bufset
            + [pltpu.VMEM((K * 8,), jnp.float32)]
            + [pltpu.SemaphoreType.DMA] * 4
        ),
        compiler_params=pltpu.CompilerParams(needs_layout_passes=False,
                                             use_tc_tiling_on_sc=False),
    )(pos8f, ii, jj)


def _combine_body(part_ref, out_ref, e_ref):
    total = part_ref[0] + part_ref[1]
    out_ref[...] = total
    rows = lax.broadcasted_iota(jnp.int32, (R128, 128), 0)
    cols = lax.broadcasted_iota(jnp.int32, (R128, 128), 1)
    is_real_ae = (rows < REAL_ROWS) & (cols % 8 == 3)
    e_ref[0, 0] = 0.5 * jnp.sum(jnp.where(is_real_ae, total, 0.0))


@jax.jit
def _combine(part):
    return pl.pallas_call(
        _combine_body,
        out_shape=(
            jax.ShapeDtypeStruct((R128, 128), jnp.float32),
            jax.ShapeDtypeStruct((1, 1), jnp.float32),
        ),
        out_specs=(
            pl.BlockSpec(memory_space=pltpu.VMEM),
            pl.BlockSpec(memory_space=pltpu.SMEM),
        ),
    )(part)


def kernel(positions, mapping):
    pos8f = jnp.pad(positions, ((0, NPAD - N_ATOMS), (0, 5))).reshape(-1)
    pad = jnp.full((P_PAD - N_PAIRS,), N_ATOMS, jnp.int32)
    ii = jnp.concatenate([mapping[0], pad])
    jj = jnp.concatenate([mapping[1], pad])
    part = _sc_call(pos8f, ii, jj)
    summed, e = _combine(part.reshape(2, R128, 128))
    flat = summed.reshape(NPAD, 8)
    forces = flat[:N_ATOMS, :3]
    atom_energies = flat[:N_ATOMS, 3]
    return (e[0, 0], atom_energies, forces)


# speedup vs baseline: 1.1267x; 1.1267x over previous
import jax
import jax.numpy as jnp
from jax import lax
from jax.experimental import pallas as pl
from jax.experimental.pallas import tpu as pltpu
from jax.experimental.pallas import tpu_sc as plsc

N_ATOMS = 100000
N_PAIRS = 6400000
NPAD = 100096
K = 128
NW = 32
CHUNKS_PER_W = 1563
NH = CHUNKS_PER_W // 3
P_PAD = NW * CHUNKS_PER_W * K
ROWS_PER_TILE = NPAD // 16
R128 = (4 * NPAD) // 128
AE_ROW0 = (3 * NPAD) // 128


def _rsqrt(x):
    i = plsc.bitcast(x, jnp.int32)
    i = 0x5F3759DF - lax.shift_right_logical(i, 1)
    y = plsc.bitcast(i, jnp.float32)
    for _ in range(2):
        y = y * (1.5 - 0.5 * x * y * y)
    return y


def _sc_body(x_hbm, y_hbm, z_hbm, zero_hbm, ii_hbm, jj_hbm, part_hbm,
             x_s, y_s, z_s, fx_s, fy_s, fz_s, ae_s,
             idx_i0, idx_j0, xi0, yi0, zi0, xj0, yj0, zj0,
             fxi0, fyi0, fzi0, fxj0, fyj0, fzj0, ev0,
             idx_i1, idx_j1, xi1, yi1, zi1, xj1, yj1, zj1,
             fxi1, fyi1, fzi1, fxj1, fyj1, fzj1, ev1,
             idx_i2, idx_j2, xi2, yi2, zi2, xj2, yj2, zj2,
             fxi2, fyi2, fzi2, fxj2, fyj2, fzj2, ev2,
             stage_v, gsem0, gsem1, gsem2, ssem0, ssem1, ssem2):
    c = lax.axis_index("c")
    s = lax.axis_index("s")

    IDX_I = (idx_i0, idx_i1, idx_i2)
    IDX_J = (idx_j0, idx_j1, idx_j2)
    GI = ((xi0, yi0, zi0), (xi1, yi1, zi1), (xi2, yi2, zi2))
    GJ = ((xj0, yj0, zj0), (xj1, yj1, zj1), (xj2, yj2, zj2))
    UPD = ((fxi0, fyi0, fzi0, fxj0, fyj0, fzj0, ev0),
           (fxi1, fyi1, fzi1, fxj1, fyj1, fzj1, ev1),
           (fxi2, fyi2, fzi2, fxj2, fyj2, fzj2, ev2))
    GSEM = (gsem0, gsem1, gsem2)
    SSEM = (ssem0, ssem1, ssem2)
    TABLES = (x_s, y_s, z_s)

    sl = pl.ds(s * ROWS_PER_TILE, ROWS_PER_TILE)
    for src, dst in ((x_hbm, x_s), (y_hbm, y_s), (z_hbm, z_s),
                     (zero_hbm, fx_s), (zero_hbm, fy_s), (zero_hbm, fz_s),
                     (zero_hbm, ae_s)):
        pltpu.sync_copy(src.at[sl], stage_v)
        pltpu.sync_copy(stage_v, dst.at[sl])
    plsc.subcore_barrier()

    w = s * 2 + c
    base = w * (CHUNKS_PER_W * K)

    def fetch(b, off):
        pltpu.sync_copy(ii_hbm.at[pl.ds(off, K)], IDX_I[b])
        pltpu.sync_copy(jj_hbm.at[pl.ds(off, K)], IDX_J[b])
        for t, dst in zip(TABLES, GI[b]):
            pltpu.async_copy(t.at[IDX_I[b]], dst, GSEM[b])
        for t, dst in zip(TABLES, GJ[b]):
            pltpu.async_copy(t.at[IDX_J[b]], dst, GSEM[b])

    def wait_gathers(b):
        for t, dst in zip(TABLES, GI[b]):
            pltpu.make_async_copy(t.at[IDX_I[b]], dst, GSEM[b]).wait()
        for t, dst in zip(TABLES, GJ[b]):
            pltpu.make_async_copy(t.at[IDX_J[b]], dst, GSEM[b]).wait()

    def _scatter_list(b):
        fxi, fyi, fzi, fxj, fyj, fzj, ev = UPD[b]
        return ((fxi, fx_s, IDX_I[b]), (fyi, fy_s, IDX_I[b]),
                (fzi, fz_s, IDX_I[b]), (ev, ae_s, IDX_I[b]),
                (fxj, fx_s, IDX_J[b]), (fyj, fy_s, IDX_J[b]),
                (fzj, fz_s, IDX_J[b]), (ev, ae_s, IDX_J[b]))

    def fire_scatters(b):
        for src, acc, idx in _scatter_list(b):
            pltpu.async_copy(src, acc.at[idx], SSEM[b], add=True)

    def wait_scatters(b):
        for src, acc, idx in _scatter_list(b):
            pltpu.make_async_copy(src, acc.at[idx], SSEM[b]).wait()

    def compute(b):
        xi_v, yi_v, zi_v = GI[b]
        xj_v, yj_v, zj_v = GJ[b]
        fxi_v, fyi_v, fzi_v, fxj_v, fyj_v, fzj_v, e_v = UPD[b]
        for grp in range(K // 16):
            o = pl.ds(grp * 16, 16)
            dx = xj_v[o] - xi_v[o]
            dy = yj_v[o] - yi_v[o]
            dz = zj_v[o] - zi_v[o]
            sq = jnp.maximum(dx * dx + dy * dy + dz * dz, 1e-24)
            yv = _rsqrt(sq)
            dist = sq * yv
            t = jnp.maximum(1.0 - dist, 0.0)
            inv_d = t * yv
            fx = inv_d * dx
            fy = inv_d * dy
            fz = inv_d * dz
            fxi_v[o] = fx
            fyi_v[o] = fy
            fzi_v[o] = fz
            fxj_v[o] = -fx
            fyj_v[o] = -fy
            fzj_v[o] = -fz
            e_v[o] = 0.25 * t * t

    fetch(0, base)

    def hbody(h, carry):
        for b in range(3):
            wait_gathers(b)
            compute(b)
            fire_scatters(b)
            nxt = (b + 1) % 3

            def _drain(nxt=nxt):
                wait_scatters(nxt)

            if b < 2:
                pl.when(h >= 1)(_drain)
                fetch(nxt, base + (3 * h + b + 1) * K)
            else:
                _drain()

                @pl.when(h < NH - 1)
                def _():
                    fetch(nxt, base + (3 * h + 3) * K)
        return carry

    lax.fori_loop(0, NH, hbody, 0)
    wait_scatters(1)
    wait_scatters(2)
    plsc.subcore_barrier()

    o0 = c * (4 * NPAD) + s * ROWS_PER_TILE
    for comp, acc in enumerate((fx_s, fy_s, fz_s, ae_s)):
        pltpu.sync_copy(acc.at[sl], stage_v)
        pltpu.sync_copy(stage_v,
                        part_hbm.at[pl.ds(o0 + comp * NPAD, ROWS_PER_TILE)])


@jax.jit
def _sc_call(x, y, z, zeros1, ii, jj):
    mesh = plsc.VectorSubcoreMesh(core_axis_name="c", subcore_axis_name="s")
    table = pltpu.VMEM_SHARED((NPAD,), jnp.float32)
    fbuf = pltpu.VMEM((K,), jnp.float32)
    ibuf = pltpu.VMEM((K,), jnp.int32)
    bufset = [ibuf, ibuf] + [fbuf] * 13
    return pl.kernel(
        _sc_body,
        out_type=jax.ShapeDtypeStruct((2 * 4 * NPAD,), jnp.float32),
        mesh=mesh,
        scratch_types=(
            [table] * 7 + bufset + bufset + bufset
            + [pltpu.VMEM((ROWS_PER_TILE,), jnp.float32)]
            + [pltpu.SemaphoreType.DMA] * 6
        ),
        compiler_params=pltpu.CompilerParams(needs_layout_passes=False),
    )(x, y, z, zeros1, ii, jj)


def _combine_body(part_ref, out_ref, e_ref):
    total = part_ref[0] + part_ref[1]
    out_ref[...] = total
    rows = lax.broadcasted_iota(jnp.int32, (R128, 128), 0)
    cols = lax.broadcasted_iota(jnp.int32, (R128, 128), 1)
    is_real_ae = (rows >= AE_ROW0) & ((rows - AE_ROW0) * 128 + cols < N_ATOMS)
    e_ref[0, 0] = 0.5 * jnp.sum(jnp.where(is_real_ae, total, 0.0))


@jax.jit
def _combine(part):
    return pl.pallas_call(
        _combine_body,
        out_shape=(
            jax.ShapeDtypeStruct((R128, 128), jnp.float32),
            jax.ShapeDtypeStruct((1, 1), jnp.float32),
        ),
        out_specs=(
            pl.BlockSpec(memory_space=pltpu.VMEM),
            pl.BlockSpec(memory_space=pltpu.SMEM),
        ),
    )(part)


def kernel(positions, mapping):
    pos_pad = jnp.pad(positions, ((0, NPAD - N_ATOMS), (0, 0)))
    x = pos_pad[:, 0]
    y = pos_pad[:, 1]
    z = pos_pad[:, 2]
    zeros1 = jnp.zeros((NPAD,), jnp.float32)
    pad = jnp.full((P_PAD - N_PAIRS,), N_ATOMS, jnp.int32)
    ii = jnp.concatenate([mapping[0], pad])
    jj = jnp.concatenate([mapping[1], pad])
    part = _sc_call(x, y, z, zeros1, ii, jj)
    summed, e = _combine(part.reshape(2, R128, 128))
    flat = summed.reshape(4, NPAD)
    forces = jnp.stack([flat[0, :N_ATOMS], flat[1, :N_ATOMS],
                        flat[2, :N_ATOMS]], axis=1)
    atom_energies = flat[3, :N_ATOMS]
    return (e[0, 0], atom_energies, forces)
